# baseline (speedup 1.0000x reference)
import functools

import jax
import jax.numpy as jnp
from jax import lax
from jax.experimental import pallas as pl
from jax.experimental.pallas import tpu as pltpu
from jax.experimental.pallas import tpu_sc as plsc

DIM = 2048
HALF = 1024
NW = 32
P_PER_W = 128
CHUNK = 16
UNROLL = 8
SLICES = DIM // 16


def _sc_body(x_hbm, yang_hbm, yin_hbm, pe_hbm, out_hbm, yang_v, yin_v, sig_v, x_v):
    c = lax.axis_index("c")
    s = lax.axis_index("s")
    wid = s * 2 + c
    s0 = wid * P_PER_W

    pltpu.sync_copy(yang_hbm, yang_v)
    pltpu.sync_copy(yin_hbm, yin_v)

    def do_chunk(m, _):
        base = s0 + m * CHUNK
        pltpu.sync_copy(pe_hbm.at[pl.ds(base, CHUNK)], sig_v)

        def sig_row(j, _):
            r = lax.rem(base + j, 12)
            r6 = lax.rem(base + j + 6, 12)

            def sig_k(k, _):
                for u in range(UNROLL):
                    off = (k * UNROLL + u) * 16
                    plsc.addupdate(sig_v.at[j, pl.ds(off, 16)],
                                   yang_v[r, pl.ds(off, 16)])
                    plsc.addupdate(sig_v.at[j, pl.ds(HALF + off, 16)],
                                   yin_v[r6, pl.ds(off, 16)])
                return 0

            lax.fori_loop(0, HALF // 16 // UNROLL, sig_k, 0)
            return 0

        lax.fori_loop(0, CHUNK, sig_row, 0)

        for b in range(4):
            pltpu.sync_copy(x_hbm.at[b, pl.ds(base, CHUNK)], x_v)

            def x_row(j, _):
                def x_k(k, _):
                    for u in range(UNROLL):
                        off = (k * UNROLL + u) * 16
                        plsc.addupdate(x_v.at[j, pl.ds(off, 16)],
                                       sig_v[j, pl.ds(off, 16)])
                    return 0

                lax.fori_loop(0, SLICES // UNROLL, x_k, 0)
                return 0

            lax.fori_loop(0, CHUNK, x_row, 0)
            pltpu.sync_copy(x_v, out_hbm.at[b, pl.ds(base, CHUNK)])
        return 0

    lax.fori_loop(0, P_PER_W // CHUNK, do_chunk, 0)


def kernel(x, yang_wheel, yin_wheel, grand_cycle_pe):
    b, s, d = x.shape
    mesh = plsc.VectorSubcoreMesh(core_axis_name="c", subcore_axis_name="s")
    k = functools.partial(
        pl.kernel,
        mesh=mesh,
        out_type=jax.ShapeDtypeStruct((b, s, d), x.dtype),
        scratch_types=[
            pltpu.VMEM(yang_wheel.shape, jnp.float32),
            pltpu.VMEM(yin_wheel.shape, jnp.float32),
            pltpu.VMEM((CHUNK, DIM), jnp.float32),
            pltpu.VMEM((CHUNK, DIM), jnp.float32),
        ],
    )(_sc_body)
    return k(x, yang_wheel, yin_wheel, grand_cycle_pe)


# speedup vs baseline: 5.0645x; 5.0645x over previous
import jax
import jax.numpy as jnp
from jax.experimental import pallas as pl
from jax.experimental.pallas import tpu as pltpu

S_TILE = 1024


def _enc_kernel(x_ref, yang_ref, yin_ref, pe_ref, o_ref):
    i = pl.program_id(0)
    base = i * S_TILE
    pos = base + jax.lax.broadcasted_iota(jnp.int32, (S_TILE, 12), 0)
    col = jax.lax.broadcasted_iota(jnp.int32, (S_TILE, 12), 1)
    yang_oh = (pos % 12 == col).astype(jnp.float32)
    yin_oh = ((pos + 6) % 12 == col).astype(jnp.float32)
    yang = jnp.dot(yang_oh, yang_ref[...], preferred_element_type=jnp.float32)
    yin = jnp.dot(yin_oh, yin_ref[...], preferred_element_type=jnp.float32)
    sig = jnp.concatenate([yang, yin], axis=-1) + pe_ref[...]
    o_ref[...] = x_ref[...] + sig[None]


def kernel(x, yang_wheel, yin_wheel, grand_cycle_pe):
    b, s, d = x.shape
    half = yang_wheel.shape[1]
    assert s % S_TILE == 0 and d == 2 * half
    n_tiles = s // S_TILE

    return pl.pallas_call(
        _enc_kernel,
        grid=(n_tiles, b),
        in_specs=[
            pl.BlockSpec((1, S_TILE, d), lambda i, j: (j, i, 0)),
            pl.BlockSpec(yang_wheel.shape, lambda i, j: (0, 0)),
            pl.BlockSpec(yin_wheel.shape, lambda i, j: (0, 0)),
            pl.BlockSpec((S_TILE, d), lambda i, j: (i, 0)),
        ],
        out_specs=pl.BlockSpec((1, S_TILE, d), lambda i, j: (j, i, 0)),
        out_shape=jax.ShapeDtypeStruct((b, s, d), x.dtype),
        compiler_params=pltpu.CompilerParams(
            dimension_semantics=("arbitrary", "arbitrary"),
        ),
    )(x, yang_wheel, yin_wheel, grand_cycle_pe)
